# Initial kernel scaffold; baseline (speedup 1.0000x reference)
import jax
import jax.numpy as jnp
from jax.experimental import pallas as pl


def kernel(table, correct_sql, student_sql):
    raise NotImplementedError("write your pallas kernel here")


# speedup vs baseline: 1.2969x; 1.2969x over previous
import functools

import jax
import jax.numpy as jnp
from jax import lax
from jax.experimental import pallas as pl
from jax.experimental.pallas import tpu as pltpu
from jax.experimental.pallas import tpu_sc as plsc

VOCAB = 1_000_000
D = 64
B = 4096
S = 200
R = 2 * S
NC, NS, L = 2, 16, 16
NW = NC * NS
BPW = B // NW
C0, C1 = 128, S - 128


def _sqrt(x):
    i = plsc.bitcast(x, jnp.int32)
    y = plsc.bitcast(jnp.int32(0x5F3759DF) - (i >> 1), jnp.float32)
    for _ in range(3):
        y = y * (1.5 - 0.5 * x * y * y)
    return x * y


def _body(table_h, cor_h, stu_h, out_h, idx_c, idx_s, rows, res, out_v,
          sem_a, sem_b):
    wid = lax.axis_index("s") * NC + lax.axis_index("c")
    base = wid * BPW
    pltpu.sync_copy(cor_h.at[pl.ds(base, BPW)], idx_c)
    pltpu.sync_copy(stu_h.at[pl.ds(base, BPW)], idx_s)

    def fire(r, slot, sem):
        buf = rows.at[slot]
        pltpu.async_copy(table_h.at[idx_c.at[r, pl.ds(0, C0)]],
                         buf.at[pl.ds(0, C0)], sem)
        pltpu.async_copy(table_h.at[idx_c.at[r, pl.ds(C0, C1)]],
                         buf.at[pl.ds(C0, C1)], sem)
        pltpu.async_copy(table_h.at[idx_s.at[r, pl.ds(0, C0)]],
                         buf.at[pl.ds(S, C0)], sem)
        pltpu.async_copy(table_h.at[idx_s.at[r, pl.ds(C0, C1)]],
                         buf.at[pl.ds(S + C0, C1)], sem)

    def drain(slot, sem):
        pltpu.make_async_copy(table_h.at[pl.ds(0, R)], rows.at[slot], sem).wait()

    def accum(slot, r):
        buf = rows.at[slot]
        z = jnp.zeros((L,), jnp.float32)

        def step_add(j, acc):
            a0, a1, a2, a3 = acc
            return (a0 + buf[j, pl.ds(0, L)], a1 + buf[j, pl.ds(L, L)],
                    a2 + buf[j, pl.ds(2 * L, L)], a3 + buf[j, pl.ds(3 * L, L)])

        def step_sub(j, acc):
            a0, a1, a2, a3 = acc
            return (a0 - buf[j, pl.ds(0, L)], a1 - buf[j, pl.ds(L, L)],
                    a2 - buf[j, pl.ds(2 * L, L)], a3 - buf[j, pl.ds(3 * L, L)])

        acc = lax.fori_loop(0, S, step_add, (z, z, z, z))
        d0, d1, d2, d3 = lax.fori_loop(S, R, step_sub, acc)
        res[r] = d0 * d0 + d1 * d1 + d2 * d2 + d3 * d3

    fire(0, 0, sem_a)

    def pair(i2, carry):
        r0 = 2 * i2
        drain(0, sem_a)
        fire(r0 + 1, 1, sem_b)
        accum(0, r0)
        drain(1, sem_b)

        @pl.when(r0 + 2 < BPW)
        def _():
            fire(r0 + 2, 0, sem_a)

        accum(1, r0 + 1)
        return carry

    lax.fori_loop(0, BPW // 2, pair, 0)

    inv = jnp.float32(1.0 / S)
    for g in range(BPW // L):
        row_ids = g * L + lax.iota(jnp.int32, L)
        sq = jnp.zeros((L,), jnp.float32)
        for k in range(L):
            col_ids = jnp.full((L,), k, jnp.int32)
            sq = sq + plsc.load_gather(res, [row_ids, col_ids])
        out_v[pl.ds(g * L, L)] = _sqrt(sq) * inv
    pltpu.sync_copy(out_v, out_h.at[pl.ds(base, BPW)])


@functools.partial(jax.jit, static_argnames=())
def _run(table, correct_sql, student_sql):
    mesh = plsc.VectorSubcoreMesh(core_axis_name="c", subcore_axis_name="s",
                                  num_cores=NC, num_subcores=NS)
    f = pl.kernel(
        _body,
        out_type=jax.ShapeDtypeStruct((B,), jnp.float32),
        mesh=mesh,
        compiler_params=pltpu.CompilerParams(needs_layout_passes=False,
                                             use_tc_tiling_on_sc=False),
        scratch_types=[
            pltpu.VMEM((BPW, S), jnp.int32),
            pltpu.VMEM((BPW, S), jnp.int32),
            pltpu.VMEM((2, R, D), jnp.float32),
            pltpu.VMEM((BPW, L), jnp.float32),
            pltpu.VMEM((BPW,), jnp.float32),
            pltpu.SemaphoreType.DMA,
            pltpu.SemaphoreType.DMA,
        ],
    )
    return f(table, correct_sql, student_sql)


def kernel(table, correct_sql, student_sql):
    return _run(table, correct_sql.astype(jnp.int32),
                student_sql.astype(jnp.int32))
